# baseline (speedup 1.0000x reference)
import jax
import jax.numpy as jnp
from jax.experimental import pallas as pl


def _shift_down(y):
    return jnp.concatenate([jnp.zeros((1, y.shape[1]), y.dtype), y[:-1]], axis=0)


def _shift_up(y):
    return jnp.concatenate([y[1:], jnp.zeros((1, y.shape[1]), y.dtype)], axis=0)


def _conv3(h, w_ref):
    y0 = jnp.dot(h, w_ref[0], preferred_element_type=jnp.float32)
    y1 = jnp.dot(h, w_ref[1], preferred_element_type=jnp.float32)
    y2 = jnp.dot(h, w_ref[2], preferred_element_type=jnp.float32)
    return _shift_down(y0) + y1 + _shift_up(y2)


def _layer_norm(h, g, b):
    m = jnp.mean(h, axis=-1, keepdims=True)
    v = jnp.mean((h - m) ** 2, axis=-1, keepdims=True)
    return (h - m) * jax.lax.rsqrt(v + 1e-5) * g + b


def _predictor(xb16, w1, b1, g1, be1, w2, b2, g2, be2, wl, bl):
    h = _conv3(xb16, w1) + b1[...]
    h = jnp.maximum(h, 0.0)
    h = _layer_norm(h, g1[...], be1[...])
    h = _conv3(h.astype(jnp.bfloat16), w2) + b2[...]
    h = jnp.maximum(h, 0.0)
    h = _layer_norm(h, g2[...], be2[...])
    return jnp.dot(h.astype(jnp.bfloat16), wl[...],
                   preferred_element_type=jnp.float32) + bl[0, 0]


def _body(x_ref, pt_ref, et_ref,
          p_w1, p_b1, p_g1, p_be1, p_w2, p_b2, p_g2, p_be2, p_wl, p_bl,
          e_w1, e_b1, e_g1, e_be1, e_w2, e_b2, e_g2, e_be2, e_wl, e_bl,
          pbins_ref, ebins_ref, pemb_ref, eemb_ref,
          xout_ref, ppred_ref, epred_ref):
    xb = x_ref[0]
    T, D = xb.shape
    xb16 = xb.astype(jnp.bfloat16)

    ppred_ref[0] = _predictor(xb16, p_w1, p_b1, p_g1, p_be1,
                              p_w2, p_b2, p_g2, p_be2, p_wl, p_bl)
    epred_ref[0] = _predictor(xb16, e_w1, e_b1, e_g1, e_be1,
                              e_w2, e_b2, e_g2, e_be2, e_wl, e_bl)

    lane = jax.lax.broadcasted_iota(jnp.int32, (T, D), 1)

    def embed_add(t_ref, bins_ref, emb_ref):
        tcol = t_ref[0, 0].reshape(T, 1)
        cnt = jnp.sum((bins_ref[...] < tcol).astype(jnp.int32), axis=1,
                      keepdims=True)
        onehot = (lane == cnt).astype(jnp.bfloat16)
        return jnp.dot(onehot, emb_ref[...], preferred_element_type=jnp.float32)

    xout_ref[0] = (xb + embed_add(pt_ref, pbins_ref, pemb_ref)
                   + embed_add(et_ref, ebins_ref, eemb_ref))


def kernel(x, pitch_target, energy_target, params):
    B, T, D = x.shape
    pp, ep = params["pitch_pred"], params["energy_pred"]

    def vec(v):
        return v.reshape(1, -1)

    pbins = jnp.full((1, D), 2.0, jnp.float32).at[0, : params["pitch_bins"].shape[0]].set(
        params["pitch_bins"])
    ebins = jnp.full((1, D), 2.0, jnp.float32).at[0, : params["energy_bins"].shape[0]].set(
        params["energy_bins"])

    grid = (B,)
    seq_spec = pl.BlockSpec((1, T, D), lambda b: (b, 0, 0))
    tgt_spec = pl.BlockSpec((1, 1, T), lambda b: (b, 0, 0))

    def full(a):
        return pl.BlockSpec(a.shape, lambda b: (0,) * a.ndim)

    bf = jnp.bfloat16
    consts = [pp["W1"].astype(bf), vec(pp["b1"]), vec(pp["g1"]), vec(pp["be1"]),
              pp["W2"].astype(bf), vec(pp["b2"]), vec(pp["g2"]), vec(pp["be2"]),
              pp["Wl"].astype(bf), pp["bl"].reshape(1, 1),
              ep["W1"].astype(bf), vec(ep["b1"]), vec(ep["g1"]), vec(ep["be1"]),
              ep["W2"].astype(bf), vec(ep["b2"]), vec(ep["g2"]), vec(ep["be2"]),
              ep["Wl"].astype(bf), ep["bl"].reshape(1, 1),
              pbins, ebins, params["pitch_embed"].astype(bf),
              params["energy_embed"].astype(bf)]

    out = pl.pallas_call(
        _body,
        grid=grid,
        in_specs=[seq_spec, tgt_spec, tgt_spec] + [full(c) for c in consts],
        out_specs=[seq_spec,
                   pl.BlockSpec((1, T, 1), lambda b: (b, 0, 0)),
                   pl.BlockSpec((1, T, 1), lambda b: (b, 0, 0))],
        out_shape=[jax.ShapeDtypeStruct((B, T, D), jnp.float32),
                   jax.ShapeDtypeStruct((B, T, 1), jnp.float32),
                   jax.ShapeDtypeStruct((B, T, 1), jnp.float32)],
    )(x, pitch_target.reshape(B, 1, T), energy_target.reshape(B, 1, T), *consts)

    x_out, ppred, epred = out
    return (x_out, ppred.reshape(B, T), epred.reshape(B, T))


# speedup vs baseline: 1.0012x; 1.0012x over previous
import jax
import jax.numpy as jnp
from jax.experimental import pallas as pl
from jax.experimental.pallas import tpu as pltpu


def _shift_down(y):
    return jnp.concatenate([jnp.zeros((1, y.shape[1]), y.dtype), y[:-1]], axis=0)


def _shift_up(y):
    return jnp.concatenate([y[1:], jnp.zeros((1, y.shape[1]), y.dtype)], axis=0)


def _conv3(h, w_ref):
    y0 = jnp.dot(h, w_ref[0], preferred_element_type=jnp.float32)
    y1 = jnp.dot(h, w_ref[1], preferred_element_type=jnp.float32)
    y2 = jnp.dot(h, w_ref[2], preferred_element_type=jnp.float32)
    return _shift_down(y0) + y1 + _shift_up(y2)


def _layer_norm(h, g, b):
    m = jnp.mean(h, axis=-1, keepdims=True)
    v = jnp.mean((h - m) ** 2, axis=-1, keepdims=True)
    return (h - m) * jax.lax.rsqrt(v + 1e-5) * g + b


def _predictor(xb16, w1, b1, g1, be1, w2, b2, g2, be2, wl, bl):
    h = _conv3(xb16, w1) + b1[...]
    h = jnp.maximum(h, 0.0)
    h = _layer_norm(h, g1[...], be1[...])
    h = _conv3(h.astype(jnp.bfloat16), w2) + b2[...]
    h = jnp.maximum(h, 0.0)
    h = _layer_norm(h, g2[...], be2[...])
    return jnp.dot(h.astype(jnp.bfloat16), wl[...],
                   preferred_element_type=jnp.float32) + bl[0, 0]


def _body(x_ref, pt_ref, et_ref,
          p_w1, p_b1, p_g1, p_be1, p_w2, p_b2, p_g2, p_be2, p_wl, p_bl,
          e_w1, e_b1, e_g1, e_be1, e_w2, e_b2, e_g2, e_be2, e_wl, e_bl,
          pbins_ref, ebins_ref, pemb_ref, eemb_ref,
          xout_ref, ppred_ref, epred_ref):
    xb = x_ref[0]
    T, D = xb.shape
    xb16 = xb.astype(jnp.bfloat16)

    ppred_ref[0] = _predictor(xb16, p_w1, p_b1, p_g1, p_be1,
                              p_w2, p_b2, p_g2, p_be2, p_wl, p_bl)
    epred_ref[0] = _predictor(xb16, e_w1, e_b1, e_g1, e_be1,
                              e_w2, e_b2, e_g2, e_be2, e_wl, e_bl)

    lane = jax.lax.broadcasted_iota(jnp.int32, (T, D), 1)

    def embed_add(t_ref, bins_ref, emb_ref):
        tcol = t_ref[0, 0].reshape(T, 1)
        cnt = jnp.sum((bins_ref[...] < tcol).astype(jnp.int32), axis=1,
                      keepdims=True)
        onehot = (lane == cnt).astype(jnp.bfloat16)
        return jnp.dot(onehot, emb_ref[...], preferred_element_type=jnp.float32)

    xout_ref[0] = (xb + embed_add(pt_ref, pbins_ref, pemb_ref)
                   + embed_add(et_ref, ebins_ref, eemb_ref))


def kernel(x, pitch_target, energy_target, params):
    B, T, D = x.shape
    pp, ep = params["pitch_pred"], params["energy_pred"]

    def vec(v):
        return v.reshape(1, -1)

    pbins = jnp.full((1, D), 2.0, jnp.float32).at[0, : params["pitch_bins"].shape[0]].set(
        params["pitch_bins"])
    ebins = jnp.full((1, D), 2.0, jnp.float32).at[0, : params["energy_bins"].shape[0]].set(
        params["energy_bins"])

    grid = (B,)
    seq_spec = pl.BlockSpec((1, T, D), lambda b: (b, 0, 0))
    tgt_spec = pl.BlockSpec((1, 1, T), lambda b: (b, 0, 0))

    def full(a):
        return pl.BlockSpec(a.shape, lambda b: (0,) * a.ndim)

    bf = jnp.bfloat16
    consts = [pp["W1"].astype(bf), vec(pp["b1"]), vec(pp["g1"]), vec(pp["be1"]),
              pp["W2"].astype(bf), vec(pp["b2"]), vec(pp["g2"]), vec(pp["be2"]),
              pp["Wl"].astype(bf), pp["bl"].reshape(1, 1),
              ep["W1"].astype(bf), vec(ep["b1"]), vec(ep["g1"]), vec(ep["be1"]),
              ep["W2"].astype(bf), vec(ep["b2"]), vec(ep["g2"]), vec(ep["be2"]),
              ep["Wl"].astype(bf), ep["bl"].reshape(1, 1),
              pbins, ebins, params["pitch_embed"].astype(bf),
              params["energy_embed"].astype(bf)]

    out = pl.pallas_call(
        _body,
        grid=grid,
        in_specs=[seq_spec, tgt_spec, tgt_spec] + [full(c) for c in consts],
        out_specs=[seq_spec,
                   pl.BlockSpec((1, T, 1), lambda b: (b, 0, 0)),
                   pl.BlockSpec((1, T, 1), lambda b: (b, 0, 0))],
        out_shape=[jax.ShapeDtypeStruct((B, T, D), jnp.float32),
                   jax.ShapeDtypeStruct((B, T, 1), jnp.float32),
                   jax.ShapeDtypeStruct((B, T, 1), jnp.float32)],
        compiler_params=pltpu.CompilerParams(
            dimension_semantics=("parallel",)),
    )(x, pitch_target.reshape(B, 1, T), energy_target.reshape(B, 1, T), *consts)

    x_out, ppred, epred = out
    return (x_out, ppred.reshape(B, T), epred.reshape(B, T))


# speedup vs baseline: 1.1072x; 1.1059x over previous
import jax
import jax.numpy as jnp
from jax.experimental import pallas as pl
from jax.experimental.pallas import tpu as pltpu


def _shift_down(y):
    rows = jax.lax.broadcasted_iota(jnp.int32, y.shape, 0)
    return jnp.where(rows == 0, 0.0, pltpu.roll(y, 1, 0))


def _shift_up(y):
    rows = jax.lax.broadcasted_iota(jnp.int32, y.shape, 0)
    return jnp.where(rows == y.shape[0] - 1, 0.0, pltpu.roll(y, y.shape[0] - 1, 0))


def _conv3(hd, h, hu, w_ref):
    y = jnp.dot(hd, w_ref[0], preferred_element_type=jnp.float32)
    y += jnp.dot(h, w_ref[1], preferred_element_type=jnp.float32)
    y += jnp.dot(hu, w_ref[2], preferred_element_type=jnp.float32)
    return y


def _layer_norm(h, g, b):
    m = jnp.mean(h, axis=-1, keepdims=True)
    v = jnp.mean((h - m) ** 2, axis=-1, keepdims=True)
    return (h - m) * jax.lax.rsqrt(v + 1e-5) * g + b


def _predictor(xd, xb, xu, w1, b1, g1, be1, w2, b2, g2, be2, wl, bl):
    h = _conv3(xd, xb, xu, w1) + b1[...]
    h = jnp.maximum(h, 0.0)
    h = _layer_norm(h, g1[...], be1[...])
    h = _conv3(_shift_down(h), h, _shift_up(h), w2) + b2[...]
    h = jnp.maximum(h, 0.0)
    h = _layer_norm(h, g2[...], be2[...])
    return jnp.dot(h, wl[...], preferred_element_type=jnp.float32) + bl[0]


def _body(x_ref, pt_ref, et_ref,
          p_w1, p_b1, p_g1, p_be1, p_w2, p_b2, p_g2, p_be2, p_wl, p_bl,
          e_w1, e_b1, e_g1, e_be1, e_w2, e_b2, e_g2, e_be2, e_wl, e_bl,
          plo_ref, phi_ref, elo_ref, ehi_ref,
          pemb_ref, eemb_ref,
          xout_ref, ppred_ref, epred_ref):
    xb = x_ref[0]
    xd, xu = _shift_down(xb), _shift_up(xb)

    ppred_ref[0] = _predictor(xd, xb, xu, p_w1, p_b1, p_g1, p_be1,
                              p_w2, p_b2, p_g2, p_be2, p_wl, p_bl)
    epred_ref[0] = _predictor(xd, xb, xu, e_w1, e_b1, e_g1, e_be1,
                              e_w2, e_b2, e_g2, e_be2, e_wl, e_bl)

    def embed_add(t_ref, lo_ref, hi_ref, emb_ref):
        v = t_ref[0]
        onehot = ((lo_ref[...] < v) & (v <= hi_ref[...])).astype(jnp.float32)
        return jnp.dot(onehot, emb_ref[...], preferred_element_type=jnp.float32)

    xout_ref[0] = (xb + embed_add(pt_ref, plo_ref, phi_ref, pemb_ref)
                   + embed_add(et_ref, elo_ref, ehi_ref, eemb_ref))


def kernel(x, pitch_target, energy_target, params):
    B, T, D = x.shape
    pp, ep = params["pitch_pred"], params["energy_pred"]

    grid = (B,)
    seq_spec = pl.BlockSpec((1, T, D), lambda b: (b, 0, 0))
    tgt_spec = pl.BlockSpec((1, T, 1), lambda b: (b, 0, 0))

    def full(a):
        return pl.BlockSpec(a.shape, lambda b: (0,) * a.ndim)

    def pad_bins(bins):
        lo = jnp.concatenate([jnp.full((1,), -1.0, jnp.float32), bins])
        hi = jnp.concatenate([bins, jnp.full((1,), 2.0, jnp.float32)])
        return lo.reshape(1, -1), hi.reshape(1, -1)

    plo, phi = pad_bins(params["pitch_bins"])
    elo, ehi = pad_bins(params["energy_bins"])

    consts = [pp["W1"], pp["b1"], pp["g1"], pp["be1"],
              pp["W2"], pp["b2"], pp["g2"], pp["be2"],
              pp["Wl"], pp["bl"],
              ep["W1"], ep["b1"], ep["g1"], ep["be1"],
              ep["W2"], ep["b2"], ep["g2"], ep["be2"],
              ep["Wl"], ep["bl"],
              plo, phi, elo, ehi,
              params["pitch_embed"], params["energy_embed"]]

    out = pl.pallas_call(
        _body,
        grid=grid,
        in_specs=[seq_spec, tgt_spec, tgt_spec] + [full(c) for c in consts],
        out_specs=[seq_spec,
                   pl.BlockSpec((1, T, 1), lambda b: (b, 0, 0)),
                   pl.BlockSpec((1, T, 1), lambda b: (b, 0, 0))],
        out_shape=[jax.ShapeDtypeStruct((B, T, D), jnp.float32),
                   jax.ShapeDtypeStruct((B, T, 1), jnp.float32),
                   jax.ShapeDtypeStruct((B, T, 1), jnp.float32)],
        compiler_params=pltpu.CompilerParams(
            dimension_semantics=("parallel",)),
    )(x, pitch_target.reshape(B, T, 1), energy_target.reshape(B, T, 1), *consts)

    x_out, ppred, epred = out
    return (x_out, ppred.reshape(B, T), epred.reshape(B, T))
